# baseline (speedup 1.0000x reference)
import jax
import jax.numpy as jnp
from jax.experimental import pallas as pl
from jax.experimental.pallas import tpu as pltpu

_B = 1024
_D = 768
_FF = 3072
_DC = _D // 4
_E = 8
_L = 6
_CAP = 160
_S = _E * _CAP

_PARAMS = pltpu.CompilerParams(vmem_limit_bytes=110 * 1024 * 1024)


def _rms_mul(h, w):
    return h * jax.lax.rsqrt(jnp.mean(h * h, axis=-1, keepdims=True) + 1e-6) * w


def _dot(a, b):
    return jnp.dot(a, b, preferred_element_type=jnp.float32)


def _proj_kernel(x_ref, w_ref, b_ref, o_ref):
    xb = x_ref[...].reshape(_B // 4, 3072)
    o_ref[...] = _dot(xb, w_ref[...]) + b_ref[...]


def _attn_block(h_ref, g1_ref, wv_ref, wo_ref, g2_ref, o_ref):
    h = h_ref[...]
    n = _rms_mul(h, g1_ref[0])
    hn = h + _dot(_dot(n, wv_ref[0]), wo_ref[0])
    o_ref[...] = hn
    return _rms_mul(hn, g2_ref[0])


def _dense_kernel(h_ref, g1_ref, wv_ref, wo_ref, g2_ref, wi_ref, w2_ref,
                  o_ref, n2_scr):
    t = pl.program_id(0)

    @pl.when(t == 0)
    def _():
        n2_scr[...] = _attn_block(h_ref, g1_ref, wv_ref, wo_ref, g2_ref,
                                  o_ref)

    @pl.when(t >= 1)
    def _():
        h1 = jnp.maximum(_dot(n2_scr[...], wi_ref[0]), 0.0)
        o_ref[...] += _dot(h1, w2_ref[0])


def _moe_kernel(h_ref, g1_ref, wv_ref, wo_ref, g2_ref, rw_ref, wia_ref,
                wib_ref, w2a_ref, w2b_ref, o_ref, ein_scr, ptg_scr):
    t = pl.program_id(0)

    @pl.when(t == 0)
    def _():
        n2 = _attn_block(h_ref, g1_ref, wv_ref, wo_ref, g2_ref, o_ref)
        logits = _dot(n2, rw_ref[0])
        col = jax.lax.broadcasted_iota(jnp.int32, (_B, _E), 1)
        m = jnp.max(logits, axis=-1, keepdims=True)
        ex = jnp.exp(logits - m)
        probs = ex / jnp.sum(ex, axis=-1, keepdims=True)
        gate = jnp.max(probs, axis=-1, keepdims=True)
        idx = jnp.min(jnp.where(probs == gate, col, _E), axis=-1,
                      keepdims=True)
        onehot = (col == idx).astype(jnp.float32)
        ri = jax.lax.broadcasted_iota(jnp.int32, (_B, _B), 0)
        ci = jax.lax.broadcasted_iota(jnp.int32, (_B, _B), 1)
        tril = (ci < ri).astype(jnp.float32)
        cnt = _dot(tril, onehot)
        pos = jnp.sum(cnt * onehot, axis=-1, keepdims=True).astype(jnp.int32)
        slot = jnp.where(pos < _CAP, idx * _CAP + pos, _S)
        scol = jax.lax.broadcasted_iota(jnp.int32, (_B, _S), 1)
        pt = (scol == slot).astype(jnp.float32)
        ptg_scr[...] = pt * gate
        ein_scr[...] = jax.lax.dot_general(
            pt, n2, (((0,), (0,)), ((), ())),
            preferred_element_type=jnp.float32)

    @pl.when((t >= 1) & (t <= _E))
    def _():
        e = t - 1
        rows = ein_scr[pl.ds(e * _CAP, _CAP), :]
        h1 = jnp.maximum(
            _dot(rows[:, :_D // 2], wia_ref[0, 0]) +
            _dot(rows[:, _D // 2:], wib_ref[0, 0]), 0.0)
        ein_scr[pl.ds(e * _CAP, _CAP), :] = (
            _dot(h1[:, :_FF // 2], w2a_ref[0, 0]) +
            _dot(h1[:, _FF // 2:], w2b_ref[0, 0]))

    @pl.when(t == _E + 1)
    def _():
        o_ref[...] += _dot(ptg_scr[...], ein_scr[...])


def _final_kernel(h_ref, g_ref, w_ref, b_ref, o_ref):
    n = _rms_mul(h_ref[...], g_ref[...])
    o_ref[...] = _dot(n, w_ref[...]) + b_ref[...]


def kernel(x, proj_W, proj_b, attn_q, attn_k, attn_v, attn_o, ln1, ln2,
           router_W, moe_wi, moe_wo, ffn_wi, ffn_wo, final_ln, fc_W, fc_b):
    f32 = jnp.float32
    sd = jax.ShapeDtypeStruct
    ln1r = ln1.reshape(_L, 1, _D)
    ln2r = ln2.reshape(_L, 1, _D)

    h = pl.pallas_call(
        _proj_kernel,
        grid=(4,),
        in_specs=[
            pl.BlockSpec((_B // 4, 3, 32, 32), lambda k: (k, 0, 0, 0)),
            pl.BlockSpec((3072, _D), lambda k: (0, 0)),
            pl.BlockSpec((1, _D), lambda k: (0, 0)),
        ],
        out_specs=pl.BlockSpec((_B // 4, _D), lambda k: (k, 0)),
        out_shape=sd((_B, _D), f32),
        compiler_params=_PARAMS)(x, proj_W, proj_b.reshape(1, _D))

    for i in range(_L):
        j = i // 2
        if i % 2 == 1:
            h = pl.pallas_call(
                _moe_kernel,
                grid=(_E + 2,),
                in_specs=[
                    pl.BlockSpec((_B, _D), lambda t: (0, 0)),
                    pl.BlockSpec((1, 1, _D), lambda t, i=i: (i, 0, 0)),
                    pl.BlockSpec((1, _D, _D), lambda t, i=i: (i, 0, 0)),
                    pl.BlockSpec((1, _D, _D), lambda t, i=i: (i, 0, 0)),
                    pl.BlockSpec((1, 1, _D), lambda t, i=i: (i, 0, 0)),
                    pl.BlockSpec((1, _D, _E), lambda t, j=j: (j, 0, 0)),
                    pl.BlockSpec(
                        (1, 1, _D // 2, _FF),
                        lambda t, j=j: (j, jnp.clip(t - 1, 0, _E - 1), 0, 0)),
                    pl.BlockSpec(
                        (1, 1, _D // 2, _FF),
                        lambda t, j=j: (j, jnp.clip(t - 1, 0, _E - 1), 1, 0)),
                    pl.BlockSpec(
                        (1, 1, _FF // 2, _D),
                        lambda t, j=j: (j, jnp.clip(t - 1, 0, _E - 1), 0, 0)),
                    pl.BlockSpec(
                        (1, 1, _FF // 2, _D),
                        lambda t, j=j: (j, jnp.clip(t - 1, 0, _E - 1), 1, 0)),
                ],
                out_specs=pl.BlockSpec((_B, _D), lambda t: (0, 0)),
                out_shape=sd((_B, _D), f32),
                scratch_shapes=[
                    pltpu.VMEM((_S, _D), f32),
                    pltpu.VMEM((_B, _S), f32),
                ],
                compiler_params=_PARAMS)(
                    h, ln1r, attn_v, attn_o, ln2r, router_W,
                    moe_wi, moe_wi, moe_wo, moe_wo)
        else:
            h = pl.pallas_call(
                _dense_kernel,
                grid=(5,),
                in_specs=[
                    pl.BlockSpec((_B, _D), lambda t: (0, 0)),
                    pl.BlockSpec((1, 1, _D), lambda t, i=i: (i, 0, 0)),
                    pl.BlockSpec((1, _D, _D), lambda t, i=i: (i, 0, 0)),
                    pl.BlockSpec((1, _D, _D), lambda t, i=i: (i, 0, 0)),
                    pl.BlockSpec((1, 1, _D), lambda t, i=i: (i, 0, 0)),
                    pl.BlockSpec((1, _D, _D),
                                 lambda t, j=j: (j, 0, jnp.clip(t - 1, 0, 3))),
                    pl.BlockSpec((1, _D, _D),
                                 lambda t, j=j: (j, jnp.clip(t - 1, 0, 3), 0)),
                ],
                out_specs=pl.BlockSpec((_B, _D), lambda t: (0, 0)),
                out_shape=sd((_B, _D), f32),
                scratch_shapes=[
                    pltpu.VMEM((_B, _D), f32),
                ],
                compiler_params=_PARAMS)(
                    h, ln1r, attn_v, attn_o, ln2r, ffn_wi, ffn_wo)

    out = pl.pallas_call(
        _final_kernel,
        out_shape=sd((_B, 10), f32),
        compiler_params=_PARAMS)(
            h, final_ln.reshape(1, _D), fc_W, fc_b.reshape(1, 10))
    return out


# speedup vs baseline: 1.1776x; 1.1776x over previous
import jax
import jax.numpy as jnp
from jax.experimental import pallas as pl
from jax.experimental.pallas import tpu as pltpu

_B = 1024
_D = 768
_FF = 3072
_DC = _D // 4
_E = 8
_L = 6
_CAP = 160
_S = _E * _CAP

_PARAMS = pltpu.CompilerParams(vmem_limit_bytes=110 * 1024 * 1024)


def _rms_mul(h, w):
    return h * jax.lax.rsqrt(jnp.mean(h * h, axis=-1, keepdims=True) + 1e-6) * w


def _dot(a, b):
    return jnp.dot(a, b, preferred_element_type=jnp.float32)


def _proj_kernel(x_ref, w_ref, b_ref, o_ref):
    o_ref[...] = _dot(x_ref[...], w_ref[...]) + b_ref[...]


def _attn_block(h_ref, g1_ref, wv_ref, wo_ref, g2_ref, o_ref):
    h = h_ref[...]
    n = _rms_mul(h, g1_ref[0])
    hn = h + _dot(_dot(n, wv_ref[0]), wo_ref[0])
    o_ref[...] = hn
    return _rms_mul(hn, g2_ref[0])


def _dense_kernel(h_ref, g1_ref, wv_ref, wo_ref, g2_ref, wi_ref, w2_ref,
                  o_ref, n2_scr):
    t = pl.program_id(0)

    @pl.when(t == 0)
    def _():
        n2_scr[...] = _attn_block(h_ref, g1_ref, wv_ref, wo_ref, g2_ref,
                                  o_ref)

    @pl.when(t >= 1)
    def _():
        h1 = jnp.maximum(_dot(n2_scr[...], wi_ref[0]), 0.0)
        o_ref[...] += _dot(h1, w2_ref[0])


def _moe_kernel(h_ref, g1_ref, wv_ref, wo_ref, g2_ref, rw_ref, wia_ref,
                wib_ref, w2a_ref, w2b_ref, o_ref, ein_scr, ptg_scr):
    t = pl.program_id(0)

    @pl.when(t == 0)
    def _():
        n2 = _attn_block(h_ref, g1_ref, wv_ref, wo_ref, g2_ref, o_ref)
        logits = _dot(n2, rw_ref[0])
        col = jax.lax.broadcasted_iota(jnp.int32, (_B, _E), 1)
        m = jnp.max(logits, axis=-1, keepdims=True)
        ex = jnp.exp(logits - m)
        probs = ex / jnp.sum(ex, axis=-1, keepdims=True)
        gate = jnp.max(probs, axis=-1, keepdims=True)
        idx = jnp.min(jnp.where(probs == gate, col, _E), axis=-1,
                      keepdims=True)
        onehot = (col == idx).astype(jnp.float32)
        ri = jax.lax.broadcasted_iota(jnp.int32, (_B, _B), 0)
        ci = jax.lax.broadcasted_iota(jnp.int32, (_B, _B), 1)
        tril = (ci < ri).astype(jnp.float32)
        cnt = _dot(tril, onehot)
        pos = jnp.sum(cnt * onehot, axis=-1, keepdims=True).astype(jnp.int32)
        slot = jnp.where(pos < _CAP, idx * _CAP + pos, _S)
        scol = jax.lax.broadcasted_iota(jnp.int32, (_B, _S), 1)
        pt = (scol == slot).astype(jnp.float32)
        ptg_scr[...] = pt * gate
        ein_scr[...] = jax.lax.dot_general(
            pt, n2, (((0,), (0,)), ((), ())),
            preferred_element_type=jnp.float32)

    @pl.when((t >= 1) & (t <= _E))
    def _():
        e = t - 1
        rows = ein_scr[pl.ds(e * _CAP, _CAP), :]
        h1 = jnp.maximum(
            _dot(rows[:, :_D // 2], wia_ref[0, 0]) +
            _dot(rows[:, _D // 2:], wib_ref[0, 0]), 0.0)
        ein_scr[pl.ds(e * _CAP, _CAP), :] = (
            _dot(h1[:, :_FF // 2], w2a_ref[0, 0]) +
            _dot(h1[:, _FF // 2:], w2b_ref[0, 0]))

    @pl.when(t == _E + 1)
    def _():
        o_ref[...] += _dot(ptg_scr[...], ein_scr[...])


def _final_kernel(h_ref, g_ref, w_ref, b_ref, o_ref):
    n = _rms_mul(h_ref[...], g_ref[...])
    o_ref[...] = _dot(n, w_ref[...]) + b_ref[...]


def kernel(x, proj_W, proj_b, attn_q, attn_k, attn_v, attn_o, ln1, ln2,
           router_W, moe_wi, moe_wo, ffn_wi, ffn_wo, final_ln, fc_W, fc_b):
    f32 = jnp.float32
    sd = jax.ShapeDtypeStruct
    xf = x.reshape(_B, -1)
    ln1r = ln1.reshape(_L, 1, _D)
    ln2r = ln2.reshape(_L, 1, _D)

    h = pl.pallas_call(
        _proj_kernel,
        grid=(4,),
        in_specs=[
            pl.BlockSpec((_B // 4, 3072), lambda k: (k, 0)),
            pl.BlockSpec((3072, _D), lambda k: (0, 0)),
            pl.BlockSpec((1, _D), lambda k: (0, 0)),
        ],
        out_specs=pl.BlockSpec((_B // 4, _D), lambda k: (k, 0)),
        out_shape=sd((_B, _D), f32),
        compiler_params=_PARAMS)(xf, proj_W, proj_b.reshape(1, _D))

    for i in range(_L):
        j = i // 2
        if i % 2 == 1:
            h = pl.pallas_call(
                _moe_kernel,
                grid=(_E + 2,),
                in_specs=[
                    pl.BlockSpec((_B, _D), lambda t: (0, 0)),
                    pl.BlockSpec((1, 1, _D), lambda t, i=i: (i, 0, 0)),
                    pl.BlockSpec((1, _D, _D), lambda t, i=i: (i, 0, 0)),
                    pl.BlockSpec((1, _D, _D), lambda t, i=i: (i, 0, 0)),
                    pl.BlockSpec((1, 1, _D), lambda t, i=i: (i, 0, 0)),
                    pl.BlockSpec((1, _D, _E), lambda t, j=j: (j, 0, 0)),
                    pl.BlockSpec(
                        (1, 1, _D // 2, _FF),
                        lambda t, j=j: (j, jnp.clip(t - 1, 0, _E - 1), 0, 0)),
                    pl.BlockSpec(
                        (1, 1, _D // 2, _FF),
                        lambda t, j=j: (j, jnp.clip(t - 1, 0, _E - 1), 1, 0)),
                    pl.BlockSpec(
                        (1, 1, _FF // 2, _D),
                        lambda t, j=j: (j, jnp.clip(t - 1, 0, _E - 1), 0, 0)),
                    pl.BlockSpec(
                        (1, 1, _FF // 2, _D),
                        lambda t, j=j: (j, jnp.clip(t - 1, 0, _E - 1), 1, 0)),
                ],
                out_specs=pl.BlockSpec((_B, _D), lambda t: (0, 0)),
                out_shape=sd((_B, _D), f32),
                scratch_shapes=[
                    pltpu.VMEM((_S, _D), f32),
                    pltpu.VMEM((_B, _S), f32),
                ],
                compiler_params=_PARAMS)(
                    h, ln1r, attn_v, attn_o, ln2r, router_W,
                    moe_wi, moe_wi, moe_wo, moe_wo)
        else:
            h = pl.pallas_call(
                _dense_kernel,
                grid=(5,),
                in_specs=[
                    pl.BlockSpec((_B, _D), lambda t: (0, 0)),
                    pl.BlockSpec((1, 1, _D), lambda t, i=i: (i, 0, 0)),
                    pl.BlockSpec((1, _D, _D), lambda t, i=i: (i, 0, 0)),
                    pl.BlockSpec((1, _D, _D), lambda t, i=i: (i, 0, 0)),
                    pl.BlockSpec((1, 1, _D), lambda t, i=i: (i, 0, 0)),
                    pl.BlockSpec((1, _D, _D),
                                 lambda t, j=j: (j, 0, jnp.clip(t - 1, 0, 3))),
                    pl.BlockSpec((1, _D, _D),
                                 lambda t, j=j: (j, jnp.clip(t - 1, 0, 3), 0)),
                ],
                out_specs=pl.BlockSpec((_B, _D), lambda t: (0, 0)),
                out_shape=sd((_B, _D), f32),
                scratch_shapes=[
                    pltpu.VMEM((_B, _D), f32),
                ],
                compiler_params=_PARAMS)(
                    h, ln1r, attn_v, attn_o, ln2r, ffn_wi, ffn_wo)

    out = pl.pallas_call(
        _final_kernel,
        out_shape=sd((_B, 10), f32),
        compiler_params=_PARAMS)(
            h, final_ln.reshape(1, _D), fc_W, fc_b.reshape(1, 10))
    return out
